# baseline (speedup 1.0000x reference)
import functools

import jax
import jax.numpy as jnp
from jax.experimental import pallas as pl
from jax.experimental.pallas import tpu as pltpu

_B = 16384
_K = 1024
_E = 32
_L = 4
_BB = 1024
_BETA = 0.25
_CH = 128


def _rvq_kernel(x_ref, cb_ref, e2_ref, xq_ref, res_ref, loss_ref, idx_ref,
                dist_ref):
    @pl.when(pl.program_id(0) == 0)
    def _init():
        loss_ref[...] = jnp.zeros((1, 1), jnp.float32)

    r = x_ref[...]
    xq = jnp.zeros_like(r)
    loss_acc = jnp.float32(0.0)
    iota_f = jax.lax.broadcasted_iota(jnp.int32, (_BB, _K), 1).astype(jnp.float32)
    lane_f = jax.lax.broadcasted_iota(jnp.int32, (_BB, _CH), 1).astype(jnp.float32)
    iota_1k = jax.lax.broadcasted_iota(jnp.int32, (1, _K), 1).astype(jnp.float32)
    for lvl in range(_L):
        emb = cb_ref[lvl]
        e2 = e2_ref[lvl]
        r2 = jnp.sum(r * r, axis=1, keepdims=True)
        ncross2 = jax.lax.dot_general(
            r, -2.0 * emb, (((1,), (1,)), ((), ())),
            preferred_element_type=jnp.float32)
        d = (r2 + e2[None, :]) + ncross2
        dist_ref[:, lvl * _K:(lvl + 1) * _K] = d
        best_v = d[:, 0:_CH]
        best_c = jnp.zeros((_BB, _CH), jnp.float32)
        for c in range(1, _K // _CH):
            v_c = d[:, c * _CH:(c + 1) * _CH]
            lt = v_c < best_v
            best_v = jnp.where(lt, v_c, best_v)
            best_c = jnp.where(lt, jnp.float32(c), best_c)
        m = jnp.min(best_v, axis=1, keepdims=True)
        cand = best_c * jnp.float32(_CH) + lane_f
        idx_f = jnp.min(jnp.where(best_v == m, cand, jnp.float32(_K)), axis=1)
        onehot = (iota_f == idx_f[:, None]).astype(jnp.float32)
        idx_row = jax.lax.dot_general(
            iota_1k, onehot, (((1,), (1,)), ((), ())),
            preferred_element_type=jnp.float32)
        idx_ref[lvl:lvl + 1, :] = idx_row.astype(jnp.int32)
        xq_raw = jax.lax.dot_general(
            onehot, emb, (((1,), (0,)), ((), ())),
            preferred_element_type=jnp.float32)
        diff = r - xq_raw
        loss_acc = loss_acc + jnp.sum(diff * diff)
        r = diff
        xq = xq + xq_raw
    xq_ref[...] = xq
    res_ref[...] = r
    loss_ref[...] += jnp.full((1, 1), loss_acc * ((1.0 + _BETA) / (_L * _B * _E)),
                              jnp.float32)


@functools.partial(jax.jit, static_argnames=())
def kernel(x, codebooks):
    grid = (_B // _BB,)
    xq, res, loss, idx_t, dist_flat = pl.pallas_call(
        _rvq_kernel,
        grid=grid,
        in_specs=[
            pl.BlockSpec((_BB, _E), lambda i: (i, 0)),
            pl.BlockSpec((_L, _K, _E), lambda i: (0, 0, 0)),
            pl.BlockSpec((_L, _K), lambda i: (0, 0)),
        ],
        out_specs=[
            pl.BlockSpec((_BB, _E), lambda i: (i, 0)),
            pl.BlockSpec((_BB, _E), lambda i: (i, 0)),
            pl.BlockSpec((1, 1), lambda i: (0, 0)),
            pl.BlockSpec((_L, _BB), lambda i: (0, i)),
            pl.BlockSpec((_BB, _L * _K), lambda i: (i, 0)),
        ],
        out_shape=[
            jax.ShapeDtypeStruct((_B, _E), jnp.float32),
            jax.ShapeDtypeStruct((_B, _E), jnp.float32),
            jax.ShapeDtypeStruct((1, 1), jnp.float32),
            jax.ShapeDtypeStruct((_L, _B), jnp.int32),
            jax.ShapeDtypeStruct((_B, _L * _K), jnp.float32),
        ],
        compiler_params=pltpu.CompilerParams(
            vmem_limit_bytes=63 * 1024 * 1024),
    )(x, codebooks, jnp.sum(codebooks * codebooks, axis=2))
    mean_losses = loss.reshape(())
    all_indices = idx_t.T
    all_distances = dist_flat.reshape(_B, _L, _K)
    return (xq, res, mean_losses, all_indices, all_distances)


# speedup vs baseline: 1.0629x; 1.0629x over previous
import functools

import jax
import jax.numpy as jnp
from jax.experimental import pallas as pl
from jax.experimental.pallas import tpu as pltpu

_B = 16384
_K = 1024
_E = 32
_L = 4
_BB = 1024
_BETA = 0.25
_CH = 128


def _rvq_kernel(x_ref, cb_ref, e2_ref, xq_ref, res_ref, loss_ref, idx_ref,
                dist_ref):
    @pl.when(pl.program_id(0) == 0)
    def _init():
        loss_ref[...] = jnp.zeros((1, 1), jnp.float32)

    r = x_ref[...]
    xq = jnp.zeros_like(r)
    loss_acc = jnp.float32(0.0)
    lane_f = jax.lax.broadcasted_iota(jnp.int32, (_BB, _CH), 1).astype(jnp.float32)
    for lvl in range(_L):
        emb = cb_ref[lvl]
        e2 = e2_ref[lvl]
        r2 = jnp.sum(r * r, axis=1, keepdims=True)
        ncross2 = jax.lax.dot_general(
            r, -2.0 * emb, (((1,), (1,)), ((), ())),
            preferred_element_type=jnp.float32)
        d = (r2 + e2[None, :]) + ncross2
        dist_ref[:, lvl * _K:(lvl + 1) * _K] = d
        best_v = d[:, 0:_CH]
        best_c = jnp.zeros((_BB, _CH), jnp.float32)
        for c in range(1, _K // _CH):
            v_c = d[:, c * _CH:(c + 1) * _CH]
            lt = v_c < best_v
            best_v = jnp.where(lt, v_c, best_v)
            best_c = jnp.where(lt, jnp.float32(c), best_c)
        m = jnp.min(best_v, axis=1, keepdims=True)
        cand = best_c * jnp.float32(_CH) + lane_f
        idx_f = jnp.min(jnp.where(best_v == m, cand, jnp.float32(_K)), axis=1)
        idx_ref[lvl:lvl + 1, :] = idx_f.astype(jnp.int32)[None, :]
        idx_c = jnp.floor(idx_f * (1.0 / _CH))
        idx_l = idx_f - idx_c * jnp.float32(_CH)
        oh_lane = (lane_f == idx_l[:, None]).astype(jnp.float32)
        xq_raw = jnp.zeros((_BB, _E), jnp.float32)
        for c in range(_K // _CH):
            y_c = jax.lax.dot_general(
                oh_lane, emb[c * _CH:(c + 1) * _CH],
                (((1,), (0,)), ((), ())),
                preferred_element_type=jnp.float32)
            ind_c = (idx_c == jnp.float32(c)).astype(jnp.float32)[:, None]
            xq_raw = xq_raw + ind_c * y_c
        diff = r - xq_raw
        loss_acc = loss_acc + jnp.sum(diff * diff)
        r = diff
        xq = xq + xq_raw
    xq_ref[...] = xq
    res_ref[...] = r
    loss_ref[...] += jnp.full((1, 1), loss_acc * ((1.0 + _BETA) / (_L * _B * _E)),
                              jnp.float32)


@functools.partial(jax.jit, static_argnames=())
def kernel(x, codebooks):
    grid = (_B // _BB,)
    xq, res, loss, idx_t, dist_flat = pl.pallas_call(
        _rvq_kernel,
        grid=grid,
        in_specs=[
            pl.BlockSpec((_BB, _E), lambda i: (i, 0)),
            pl.BlockSpec((_L, _K, _E), lambda i: (0, 0, 0)),
            pl.BlockSpec((_L, _K), lambda i: (0, 0)),
        ],
        out_specs=[
            pl.BlockSpec((_BB, _E), lambda i: (i, 0)),
            pl.BlockSpec((_BB, _E), lambda i: (i, 0)),
            pl.BlockSpec((1, 1), lambda i: (0, 0)),
            pl.BlockSpec((_L, _BB), lambda i: (0, i)),
            pl.BlockSpec((_BB, _L * _K), lambda i: (i, 0)),
        ],
        out_shape=[
            jax.ShapeDtypeStruct((_B, _E), jnp.float32),
            jax.ShapeDtypeStruct((_B, _E), jnp.float32),
            jax.ShapeDtypeStruct((1, 1), jnp.float32),
            jax.ShapeDtypeStruct((_L, _B), jnp.int32),
            jax.ShapeDtypeStruct((_B, _L * _K), jnp.float32),
        ],
        compiler_params=pltpu.CompilerParams(
            vmem_limit_bytes=63 * 1024 * 1024),
    )(x, codebooks, jnp.sum(codebooks * codebooks, axis=2))
    mean_losses = loss.reshape(())
    all_indices = idx_t.T
    all_distances = dist_flat.reshape(_B, _L, _K)
    return (xq, res, mean_losses, all_indices, all_distances)
